# Initial kernel scaffold; baseline (speedup 1.0000x reference)
import jax
import jax.numpy as jnp
from jax.experimental import pallas as pl


def kernel(propensity_type_ids, propensity_params, type_table, W_param, b_param, W_out, b_out):
    raise NotImplementedError("write your pallas kernel here")


# speedup vs baseline: 2.7533x; 2.7533x over previous
import functools

import jax
import jax.numpy as jnp
from jax import lax
from jax.experimental import pallas as pl
from jax.experimental.pallas import tpu as pltpu
from jax.experimental.pallas import tpu_sc as plsc

_LW = 128


def _sc_gather(ids3d, table):
    nw_dim, idxrows_per_w_dim, lw = ids3d.shape
    nrows = nw_dim * idxrows_per_w_dim
    v, h = table.shape
    n = nrows * lw
    info = plsc.get_sparse_core_info()
    nw = info.num_cores * info.num_subcores
    assert nw == nw_dim
    idxrows_per_w = idxrows_per_w_dim
    ch = 5
    nch = idxrows_per_w // ch
    rows_per_chunk = ch * lw
    rows_per_w = idxrows_per_w * lw
    assert nch * ch == idxrows_per_w

    mesh = plsc.VectorSubcoreMesh(core_axis_name="c", subcore_axis_name="s")

    @functools.partial(
        pl.kernel,
        out_type=jax.ShapeDtypeStruct((n, h), jnp.float32),
        mesh=mesh,
        scratch_types=[
            pltpu.VMEM((idxrows_per_w, lw), jnp.int32),
            pltpu.VMEM((rows_per_chunk, h), jnp.float32),
            pltpu.SemaphoreType.DMA,
        ],
        compiler_params=pltpu.CompilerParams(use_tc_tiling_on_sc=False),
    )
    def k(ids_hbm, table_hbm, out_hbm, idx_v, rows_v, sem):
        wid = lax.axis_index("s") * info.num_cores + lax.axis_index("c")
        row_base = wid * rows_per_w
        pltpu.sync_copy(ids_hbm.at[wid], idx_v)

        def body(c, carry):
            copies = [
                pltpu.async_copy(
                    table_hbm.at[idx_v.at[c * ch + j]],
                    rows_v.at[pl.ds(j * lw, lw)],
                    sem,
                )
                for j in range(ch)
            ]
            for cp in copies:
                cp.wait()
            out_off = pl.multiple_of(row_base + c * rows_per_chunk, 8)
            pltpu.sync_copy(rows_v, out_hbm.at[pl.ds(out_off, rows_per_chunk)])
            return carry

        lax.fori_loop(0, nch, body, 0)

    return k(ids3d, table)


def _tc_combine(gathered, params, w_param, b_param, w_out, b_out, tn=2048):
    n, h = gathered.shape
    _, p = params.shape
    d = w_out.shape[0]
    assert n % tn == 0

    def body(g_ref, p_ref, wp_ref, bp_ref, wo_ref, bo_ref, o_ref):
        g = g_ref[...]
        pr = p_ref[...]
        wo = wo_ref[...]
        dn = (((1,), (1,)), ((), ()))
        pe = lax.dot_general(pr, wp_ref[...], dn,
                             preferred_element_type=jnp.float32) + bp_ref[...]
        acc = lax.dot_general(g, wo[:, :h], dn, preferred_element_type=jnp.float32)
        acc = acc + lax.dot_general(pe, wo[:, h:], dn,
                                    preferred_element_type=jnp.float32)
        o_ref[...] = acc + bo_ref[...]

    return pl.pallas_call(
        body,
        grid=(n // tn,),
        in_specs=[
            pl.BlockSpec((tn, h), lambda i: (i, 0)),
            pl.BlockSpec((tn, p), lambda i: (i, 0)),
            pl.BlockSpec((h, p), lambda i: (0, 0)),
            pl.BlockSpec((1, h), lambda i: (0, 0)),
            pl.BlockSpec((d, d), lambda i: (0, 0)),
            pl.BlockSpec((1, d), lambda i: (0, 0)),
        ],
        out_specs=pl.BlockSpec((tn, d), lambda i: (i, 0)),
        out_shape=jax.ShapeDtypeStruct((n, d), jnp.float32),
    )(gathered, params, w_param, b_param, w_out, b_out)


def kernel(propensity_type_ids, propensity_params, type_table, W_param, b_param, W_out, b_out):
    b, r = propensity_type_ids.shape
    _, _, p = propensity_params.shape
    v, h = type_table.shape
    d = W_out.shape[0]
    n = b * r
    info = plsc.get_sparse_core_info()
    nw = info.num_cores * info.num_subcores
    ids3d = propensity_type_ids.reshape(nw, n // (nw * _LW), _LW).astype(jnp.int32)
    gathered = _sc_gather(ids3d, type_table)
    out = _tc_combine(
        gathered,
        propensity_params.reshape(n, p),
        W_param,
        b_param.reshape(1, h),
        W_out,
        b_out.reshape(1, d),
    )
    return out.reshape(b, r, d)


# speedup vs baseline: 3.1074x; 1.1286x over previous
import functools

import jax
import jax.numpy as jnp
from jax import lax
from jax.experimental import pallas as pl
from jax.experimental.pallas import tpu as pltpu
from jax.experimental.pallas import tpu_sc as plsc

_LW = 128


def _tc_prepare(table, w_param, b_param, w_out, b_out):
    v, h = table.shape
    d = w_out.shape[0]
    p = w_param.shape[1]

    def body(t_ref, wp_ref, bp_ref, wo_ref, bo_ref, tp_ref, wc_ref):
        wo = wo_ref[...]
        w1 = wo[:, :h]
        w2 = wo[:, h:]
        dn_t = (((1,), (1,)), ((), ()))
        b_eff = lax.dot_general(bp_ref[...], w2, dn_t,
                                preferred_element_type=jnp.float32) + bo_ref[...]
        tp_ref[...] = lax.dot_general(t_ref[...], w1, dn_t,
                                      preferred_element_type=jnp.float32) + b_eff
        wc_ref[...] = lax.dot_general(wp_ref[...], w2,
                                      (((0,), (1,)), ((), ())),
                                      preferred_element_type=jnp.float32)

    return pl.pallas_call(
        body,
        out_shape=(
            jax.ShapeDtypeStruct((v, d), jnp.float32),
            jax.ShapeDtypeStruct((p, d), jnp.float32),
        ),
    )(table, w_param, b_param, w_out, b_out)


def _sc_gather(ids3d, table_proj):
    nw_dim, idxrows_per_w, lw = ids3d.shape
    nrows = nw_dim * idxrows_per_w
    v, d = table_proj.shape
    n = nrows * lw
    info = plsc.get_sparse_core_info()
    nw = info.num_cores * info.num_subcores
    assert nw == nw_dim
    ch = 5
    nch = idxrows_per_w // ch
    rows_per_chunk = ch * lw
    rows_per_w = idxrows_per_w * lw
    assert nch * ch == idxrows_per_w

    mesh = plsc.VectorSubcoreMesh(core_axis_name="c", subcore_axis_name="s")

    @functools.partial(
        pl.kernel,
        out_type=jax.ShapeDtypeStruct((n, d), jnp.float32),
        mesh=mesh,
        scratch_types=[
            pltpu.VMEM((idxrows_per_w, lw), jnp.int32),
            pltpu.VMEM((rows_per_chunk, d), jnp.float32),
            pltpu.SemaphoreType.DMA,
        ],
    )
    def k(ids_hbm, table_hbm, out_hbm, idx_v, rows_v, sem):
        wid = lax.axis_index("s") * info.num_cores + lax.axis_index("c")
        row_base = wid * rows_per_w
        pltpu.sync_copy(ids_hbm.at[wid], idx_v)

        def body(c, carry):
            copies = [
                pltpu.async_copy(
                    table_hbm.at[idx_v.at[c * ch + j]],
                    rows_v.at[pl.ds(j * lw, lw)],
                    sem,
                )
                for j in range(ch)
            ]
            for cp in copies:
                cp.wait()
            out_off = pl.multiple_of(row_base + c * rows_per_chunk, 8)
            pltpu.sync_copy(rows_v, out_hbm.at[pl.ds(out_off, rows_per_chunk)])
            return carry

        lax.fori_loop(0, nch, body, 0)

    return k(ids3d, table_proj)


def _tc_combine(gathered, params, wc_t, tn=2048):
    n, d = gathered.shape
    _, p = params.shape
    assert n % tn == 0

    def body(g_ref, p_ref, wc_ref, o_ref):
        pe = lax.dot_general(p_ref[...], wc_ref[...], (((1,), (0,)), ((), ())),
                             preferred_element_type=jnp.float32)
        o_ref[...] = g_ref[...] + pe

    return pl.pallas_call(
        body,
        grid=(n // tn,),
        in_specs=[
            pl.BlockSpec((tn, d), lambda i: (i, 0)),
            pl.BlockSpec((tn, p), lambda i: (i, 0)),
            pl.BlockSpec((p, d), lambda i: (0, 0)),
        ],
        out_specs=pl.BlockSpec((tn, d), lambda i: (i, 0)),
        out_shape=jax.ShapeDtypeStruct((n, d), jnp.float32),
    )(gathered, params, wc_t)


def kernel(propensity_type_ids, propensity_params, type_table, W_param, b_param, W_out, b_out):
    b, r = propensity_type_ids.shape
    _, _, p = propensity_params.shape
    v, h = type_table.shape
    d = W_out.shape[0]
    n = b * r
    table_proj, wc_t = _tc_prepare(
        type_table, W_param, b_param.reshape(1, h), W_out, b_out.reshape(1, d)
    )
    info = plsc.get_sparse_core_info()
    nw = info.num_cores * info.num_subcores
    ids3d = propensity_type_ids.reshape(nw, n // (nw * _LW), _LW).astype(jnp.int32)
    gathered = _sc_gather(ids3d, table_proj)
    out = _tc_combine(gathered, propensity_params.reshape(n, p), wc_t)
    return out.reshape(b, r, d)


# speedup vs baseline: 3.1756x; 1.0220x over previous
import functools

import jax
import jax.numpy as jnp
from jax import lax
from jax.experimental import pallas as pl
from jax.experimental.pallas import tpu as pltpu
from jax.experimental.pallas import tpu_sc as plsc

_LW = 128


def _tc_prepare(table, w_param, b_param, w_out, b_out):
    v, h = table.shape
    d = w_out.shape[0]
    p = w_param.shape[1]

    def body(t_ref, wp_ref, bp_ref, wo_ref, bo_ref, tp_ref, wc_ref):
        wo = wo_ref[...]
        w1 = wo[:, :h]
        w2 = wo[:, h:]
        dn_t = (((1,), (1,)), ((), ()))
        b_eff = lax.dot_general(bp_ref[...], w2, dn_t,
                                preferred_element_type=jnp.float32) + bo_ref[...]
        tp_ref[...] = lax.dot_general(t_ref[...], w1, dn_t,
                                      preferred_element_type=jnp.float32) + b_eff
        wc_ref[...] = lax.dot_general(wp_ref[...], w2,
                                      (((0,), (1,)), ((), ())),
                                      preferred_element_type=jnp.float32)

    return pl.pallas_call(
        body,
        out_shape=(
            jax.ShapeDtypeStruct((v, d), jnp.float32),
            jax.ShapeDtypeStruct((p, d), jnp.float32),
        ),
    )(table, w_param, b_param, w_out, b_out)


def _sc_gather(ids3d, table_proj):
    nw_dim, idxrows_per_w, lw = ids3d.shape
    nrows = nw_dim * idxrows_per_w
    v, d = table_proj.shape
    n = nrows * lw
    info = plsc.get_sparse_core_info()
    nw = info.num_cores * info.num_subcores
    assert nw == nw_dim
    nch = idxrows_per_w
    rows_per_chunk = lw
    rows_per_w = idxrows_per_w * lw

    mesh = plsc.VectorSubcoreMesh(core_axis_name="c", subcore_axis_name="s")

    nbuf = 5
    assert nch % nbuf == 0

    @functools.partial(
        pl.kernel,
        out_type=jax.ShapeDtypeStruct((n, d), jnp.float32),
        mesh=mesh,
        scratch_types=[
            pltpu.VMEM((idxrows_per_w, lw), jnp.int32),
            [pltpu.VMEM((rows_per_chunk, d), jnp.float32) for _ in range(nbuf)],
            [pltpu.SemaphoreType.DMA for _ in range(nbuf)],
            [pltpu.SemaphoreType.DMA for _ in range(nbuf)],
        ],
    )
    def k(ids_hbm, table_hbm, out_hbm, idx_v, bufs, gsems, osems):
        wid = lax.axis_index("s") * info.num_cores + lax.axis_index("c")
        row_base = wid * rows_per_w
        pltpu.sync_copy(ids_hbm.at[wid], idx_v)

        def gather_descr(c, b, make_only):
            ctor = pltpu.make_async_copy if make_only else pltpu.async_copy
            return ctor(table_hbm.at[idx_v.at[c]], bufs[b], gsems[b])

        def out_descr(c, b, make_only):
            ctor = pltpu.make_async_copy if make_only else pltpu.async_copy
            out_off = pl.multiple_of(row_base + c * rows_per_chunk, 8)
            return ctor(bufs[b], out_hbm.at[pl.ds(out_off, rows_per_chunk)], osems[b])

        for b in range(nbuf):
            gather_descr(b, b, False)

        def body(s, carry):
            c0 = s * nbuf
            for b in range(nbuf):
                gather_descr(c0 + b, b, True).wait()
                out_descr(c0 + b, b, False)
            for b in range(nbuf):
                out_descr(c0 + b, b, True).wait()

                @pl.when(c0 + b + nbuf < nch)
                def _():
                    gather_descr(c0 + b + nbuf, b, False)

            return carry

        lax.fori_loop(0, nch // nbuf, body, 0)

    return k(ids3d, table_proj)


def _tc_combine(gathered, params3d, wc_t, tb=16):
    n, d = gathered.shape
    b, r, p = params3d.shape
    assert b % tb == 0 and tb * r * (b // tb) == n
    tn = tb * r

    def body(g_ref, p_ref, wc_ref, o_ref):
        pr = p_ref[...].reshape(tn, p)
        pe = lax.dot_general(pr, wc_ref[...], (((1,), (0,)), ((), ())),
                             preferred_element_type=jnp.float32)
        o_ref[...] = g_ref[...] + pe

    return pl.pallas_call(
        body,
        grid=(b // tb,),
        in_specs=[
            pl.BlockSpec((tn, d), lambda i: (i, 0)),
            pl.BlockSpec((tb, r, p), lambda i: (i, 0, 0)),
            pl.BlockSpec((p, d), lambda i: (0, 0)),
        ],
        out_specs=pl.BlockSpec((tn, d), lambda i: (i, 0)),
        out_shape=jax.ShapeDtypeStruct((n, d), jnp.float32),
    )(gathered, params3d, wc_t)


def kernel(propensity_type_ids, propensity_params, type_table, W_param, b_param, W_out, b_out):
    b, r = propensity_type_ids.shape
    _, _, p = propensity_params.shape
    v, h = type_table.shape
    d = W_out.shape[0]
    n = b * r
    table_proj, wc_t = _tc_prepare(
        type_table, W_param, b_param.reshape(1, h), W_out, b_out.reshape(1, d)
    )
    info = plsc.get_sparse_core_info()
    nw = info.num_cores * info.num_subcores
    ids3d = propensity_type_ids.reshape(nw, n // (nw * _LW), _LW).astype(jnp.int32)
    gathered = _sc_gather(ids3d, table_proj)
    out = _tc_combine(gathered, propensity_params, wc_t)
    return out.reshape(b, r, d)


# speedup vs baseline: 3.5322x; 1.1123x over previous
import functools

import jax
import jax.numpy as jnp
from jax import lax
from jax.experimental import pallas as pl
from jax.experimental.pallas import tpu as pltpu
from jax.experimental.pallas import tpu_sc as plsc

_LW = 128
_NSLICE = 2
_TN = 4096


def _tc_prepare(table, w_param, b_param, w_out, b_out):
    v, h = table.shape
    d = w_out.shape[0]
    p = w_param.shape[1]

    def body(t_ref, wp_ref, bp_ref, wo_ref, bo_ref, tp_ref, wc_ref):
        wo = wo_ref[...]
        w1 = wo[:, :h]
        w2 = wo[:, h:]
        dn_t = (((1,), (1,)), ((), ()))
        b_eff = lax.dot_general(bp_ref[...], w2, dn_t,
                                preferred_element_type=jnp.float32) + bo_ref[...]
        tp_ref[...] = lax.dot_general(t_ref[...], w1, dn_t,
                                      preferred_element_type=jnp.float32) + b_eff
        wc_ref[...] = lax.dot_general(wp_ref[...], w2,
                                      (((0,), (1,)), ((), ())),
                                      preferred_element_type=jnp.float32)

    return pl.pallas_call(
        body,
        out_shape=(
            jax.ShapeDtypeStruct((v, d), jnp.float32),
            jax.ShapeDtypeStruct((p, d), jnp.float32),
        ),
    )(table, w_param, b_param, w_out, b_out)


def _sc_gather(ids3d, table_proj):
    nw_dim, idxrows_per_w, lw = ids3d.shape
    v, d = table_proj.shape
    n = nw_dim * idxrows_per_w * lw
    info = plsc.get_sparse_core_info()
    nw = info.num_cores * info.num_subcores
    assert nw == nw_dim
    ch = 5
    nch = idxrows_per_w // ch
    rows_per_chunk = ch * lw
    rows_per_w = idxrows_per_w * lw
    assert nch * ch == idxrows_per_w

    mesh = plsc.VectorSubcoreMesh(core_axis_name="c", subcore_axis_name="s")

    @functools.partial(
        pl.kernel,
        out_type=jax.ShapeDtypeStruct((n, d), jnp.float32),
        mesh=mesh,
        scratch_types=[
            pltpu.VMEM((idxrows_per_w, lw), jnp.int32),
            pltpu.VMEM((rows_per_chunk, d), jnp.float32),
            pltpu.SemaphoreType.DMA,
        ],
    )
    def k(ids_hbm, table_hbm, out_hbm, idx_v, rows_v, sem):
        wid = lax.axis_index("s") * info.num_cores + lax.axis_index("c")
        row_base = wid * rows_per_w
        pltpu.sync_copy(ids_hbm.at[wid], idx_v)

        def body(c, carry):
            copies = [
                pltpu.async_copy(
                    table_hbm.at[idx_v.at[c * ch + j]],
                    rows_v.at[pl.ds(j * lw, lw)],
                    sem,
                )
                for j in range(ch)
            ]
            for cp in copies:
                cp.wait()
            out_off = pl.multiple_of(row_base + c * rows_per_chunk, 8)
            pltpu.sync_copy(rows_v, out_hbm.at[pl.ds(out_off, rows_per_chunk)])
            return carry

        lax.fori_loop(0, nch, body, 0)

    return k(ids3d, table_proj)


def _tc_combine_slice(gathered_s, params_2d, wc_t, prev, s, n, tn=_TN):
    ns, d = gathered_s.shape
    p = wc_t.shape[0]
    nblk = ns // tn
    blk0 = s * nblk
    assert nblk * tn == ns

    def body(g_ref, pk_ref, wc_ref, *o_refs):
        o_ref = o_refs[-1]
        pe = lax.dot_general(pk_ref[...], wc_ref[...], (((1,), (0,)), ((), ())),
                             preferred_element_type=jnp.float32)
        o_ref[...] = g_ref[...] + pe

    in_specs = [
        pl.BlockSpec((tn, d), lambda i: (i, 0)),
        pl.BlockSpec((tn, p), lambda i: (blk0 + i, 0)),
        pl.BlockSpec((p, d), lambda i: (0, 0)),
    ]
    args = [gathered_s, params_2d, wc_t]
    aliases = {}
    if prev is not None:
        in_specs.append(pl.BlockSpec(memory_space=pl.ANY))
        args.append(prev)
        aliases = {3: 0}

    return pl.pallas_call(
        body,
        grid=(nblk,),
        in_specs=in_specs,
        out_specs=pl.BlockSpec((tn, d), lambda i: (blk0 + i, 0)),
        out_shape=jax.ShapeDtypeStruct((n, d), jnp.float32),
        input_output_aliases=aliases,
    )(*args)


def kernel(propensity_type_ids, propensity_params, type_table, W_param, b_param, W_out, b_out):
    b, r = propensity_type_ids.shape
    _, _, p = propensity_params.shape
    v, h = type_table.shape
    d = W_out.shape[0]
    n = b * r
    table_proj, wc_t = _tc_prepare(
        type_table, W_param, b_param.reshape(1, h), W_out, b_out.reshape(1, d)
    )
    info = plsc.get_sparse_core_info()
    nw = info.num_cores * info.num_subcores
    ns = n // _NSLICE
    ids4d = propensity_type_ids.reshape(
        _NSLICE, nw, ns // (nw * _LW), _LW
    ).astype(jnp.int32)
    params_2d = propensity_params.reshape(n, p)

    gathered = [_sc_gather(ids4d[s], table_proj) for s in range(_NSLICE)]
    out = None
    for s in range(_NSLICE):
        out = _tc_combine_slice(gathered[s], params_2d, wc_t, out, s, n)
    return out.reshape(b, r, d)
